# baseline (speedup 1.0000x reference)
import functools

import jax
import jax.numpy as jnp
from jax import lax
from jax.experimental import pallas as pl
from jax.experimental.pallas import tpu as pltpu
from jax.experimental.pallas import tpu_sc as plsc

N_NODES = 10000
IN_FEATS = 128
HIDDEN = 128
NUM_CLASSES = 64
N_EDGES = 320000

NC = 2
NS = 16
L = 16
NW = NC * NS
E_PER_W = N_EDGES // NW
CHUNK = 128
E_PER_W_PAD = 10240
N_CHUNKS = E_PER_W_PAD // CHUNK
E_TOTAL_PAD = NW * E_PER_W_PAD
N_PAD = 10112
ROWS_PER_TILE = N_PAD // NS
N_FEAT_REGS = HIDDEN // L


def _spmm_body(y_hbm, rows_hbm, cols_hbm, vals_hbm, out_hbm,
               rowb, colb, valb, gbuf, shared, semg0, semg1, semi):
    cid = lax.axis_index("c")
    tid = lax.axis_index("s")
    wid = cid * NS + tid

    zero = jnp.zeros((L,), jnp.float32)

    def zero_row(i, c):
        for j in range(N_FEAT_REGS):
            gbuf[0, i, pl.ds(j * L, L)] = zero
        return c

    lax.fori_loop(0, CHUNK, zero_row, 0)
    base = tid * ROWS_PER_TILE
    done = 0
    while done < ROWS_PER_TILE:
        n = min(CHUNK, ROWS_PER_TILE - done)
        pltpu.sync_copy(gbuf.at[0, pl.ds(0, n)],
                        shared.at[pl.ds(base + done, n)])
        done += n

    def stage_idx(k, b):
        ci = pltpu.async_copy(cols_hbm.at[wid, k], colb.at[b], semi)
        ri = pltpu.async_copy(rows_hbm.at[wid, k], rowb.at[b], semi)
        vi = pltpu.async_copy(vals_hbm.at[wid, k], valb.at[b], semi)
        ci.wait()
        ri.wait()
        vi.wait()

    def start_gather(b):
        sem = semg0 if b == 0 else semg1
        pltpu.async_copy(y_hbm.at[colb.at[b]], gbuf.at[b], sem)

    def wait_gather(b):
        sem = semg0 if b == 0 else semg1
        pltpu.make_async_copy(y_hbm.at[colb.at[b]], gbuf.at[b], sem).wait()

    def scale_scatter(b):
        def scale_group(g, c2):
            vvec = valb[b, pl.ds(g * L, L)]
            for li in range(L):
                v = vvec[li]
                i = g * L + li
                for j in range(N_FEAT_REGS):
                    sl = pl.ds(j * L, L)
                    gbuf[b, i, sl] = gbuf[b, i, sl] * v
            return c2

        lax.fori_loop(0, CHUNK // L, scale_group, 0)
        pltpu.sync_copy(gbuf.at[b], shared.at[rowb.at[b]], add=True)

    stage_idx(0, 0)
    start_gather(0)
    stage_idx(1, 1)
    plsc.subcore_barrier()

    def pair_body(p, c):
        k = 2 * p
        wait_gather(0)
        start_gather(1)
        scale_scatter(0)

        @pl.when(k + 2 < N_CHUNKS)
        def _():
            stage_idx(k + 2, 0)

        wait_gather(1)

        @pl.when(k + 2 < N_CHUNKS)
        def _():
            start_gather(0)

        scale_scatter(1)

        @pl.when(k + 3 < N_CHUNKS)
        def _():
            stage_idx(k + 3, 1)

        return c

    lax.fori_loop(0, N_CHUNKS // 2, pair_body, 0)

    plsc.subcore_barrier()
    pltpu.sync_copy(
        shared.at[pl.ds(base, ROWS_PER_TILE)],
        out_hbm.at[cid, pl.ds(base, ROWS_PER_TILE)])


_spmm_call = pl.kernel(
    _spmm_body,
    out_type=jax.ShapeDtypeStruct((NC, N_PAD, HIDDEN), jnp.float32),
    mesh=plsc.VectorSubcoreMesh(
        core_axis_name="c", subcore_axis_name="s",
        num_cores=NC, num_subcores=NS),
    scratch_types=[
        pltpu.VMEM((2, CHUNK), jnp.int32),
        pltpu.VMEM((2, CHUNK), jnp.int32),
        pltpu.VMEM((2, CHUNK), jnp.float32),
        pltpu.VMEM((2, CHUNK, HIDDEN), jnp.float32),
        pltpu.VMEM_SHARED((N_PAD, HIDDEN), jnp.float32),
        pltpu.SemaphoreType.DMA,
        pltpu.SemaphoreType.DMA,
        pltpu.SemaphoreType.DMA,
    ],
)


MBLK = 400


def _lin1_body(x_ref, w_ref, b_ref, o_ref):
    o_ref[...] = (
        jnp.dot(x_ref[...], w_ref[...], preferred_element_type=jnp.float32)
        + b_ref[...])


def _lin2_body(p0_ref, p1_ref, w_ref, b_ref, o_ref):
    h = jnp.maximum(p0_ref[...] + p1_ref[...], 0.0)
    o_ref[...] = (
        jnp.dot(h, w_ref[...], preferred_element_type=jnp.float32)
        + b_ref[...])


def _head_body(q0_ref, q1_ref, wc_ref, bc_ref, o_ref, acc_ref):
    i = pl.program_id(0)

    @pl.when(i == 0)
    def _():
        acc_ref[...] = jnp.zeros_like(acc_ref)

    h = jnp.maximum(q0_ref[...] + q1_ref[...], 0.0)
    acc_ref[...] += jnp.sum(h, axis=0, keepdims=True)

    @pl.when(i == pl.num_programs(0) - 1)
    def _():
        pooled = acc_ref[...] * (1.0 / N_NODES)
        o_ref[...] = (
            jnp.dot(pooled, wc_ref[...], preferred_element_type=jnp.float32)
            + bc_ref[...])


def _linear1(x, W, b):
    return pl.pallas_call(
        _lin1_body,
        grid=(N_NODES // MBLK,),
        in_specs=[
            pl.BlockSpec((MBLK, IN_FEATS), lambda i: (i, 0)),
            pl.BlockSpec((IN_FEATS, HIDDEN), lambda i: (0, 0)),
            pl.BlockSpec((1, HIDDEN), lambda i: (0, 0)),
        ],
        out_specs=pl.BlockSpec((MBLK, HIDDEN), lambda i: (i, 0)),
        out_shape=jax.ShapeDtypeStruct((N_NODES, HIDDEN), jnp.float32),
    )(x, W, b.reshape(1, HIDDEN))


def _linear2(p0, p1, W, b):
    return pl.pallas_call(
        _lin2_body,
        grid=(N_NODES // MBLK,),
        in_specs=[
            pl.BlockSpec((MBLK, HIDDEN), lambda i: (i, 0)),
            pl.BlockSpec((MBLK, HIDDEN), lambda i: (i, 0)),
            pl.BlockSpec((HIDDEN, HIDDEN), lambda i: (0, 0)),
            pl.BlockSpec((1, HIDDEN), lambda i: (0, 0)),
        ],
        out_specs=pl.BlockSpec((MBLK, HIDDEN), lambda i: (i, 0)),
        out_shape=jax.ShapeDtypeStruct((N_NODES, HIDDEN), jnp.float32),
    )(p0, p1, W, b.reshape(1, HIDDEN))


def _head(q0, q1, Wc, bc):
    out = pl.pallas_call(
        _head_body,
        grid=(N_NODES // MBLK,),
        in_specs=[
            pl.BlockSpec((MBLK, HIDDEN), lambda i: (i, 0)),
            pl.BlockSpec((MBLK, HIDDEN), lambda i: (i, 0)),
            pl.BlockSpec((HIDDEN, NUM_CLASSES), lambda i: (0, 0)),
            pl.BlockSpec((1, NUM_CLASSES), lambda i: (0, 0)),
        ],
        out_specs=pl.BlockSpec((1, NUM_CLASSES), lambda i: (0, 0)),
        out_shape=jax.ShapeDtypeStruct((1, NUM_CLASSES), jnp.float32),
        scratch_shapes=[pltpu.VMEM((1, HIDDEN), jnp.float32)],
    )(q0, q1, Wc, bc.reshape(1, NUM_CLASSES))
    return out.reshape(NUM_CLASSES)


@jax.jit
def kernel(x, adj_indices, adj_values, W1, b1, W2, b2, Wc, bc):
    pad = E_TOTAL_PAD - N_EDGES
    rows = jnp.concatenate(
        [adj_indices[0].astype(jnp.int32),
         jnp.full((pad,), N_NODES, jnp.int32)]).reshape(NW, N_CHUNKS, CHUNK)
    cols = jnp.concatenate(
        [adj_indices[1].astype(jnp.int32),
         jnp.zeros((pad,), jnp.int32)]).reshape(NW, N_CHUNKS, CHUNK)
    vals = jnp.concatenate(
        [adj_values, jnp.zeros((pad,), jnp.float32)]
    ).reshape(NW, N_CHUNKS, CHUNK)

    y1 = _linear1(x, W1, b1)
    p = _spmm_call(y1, rows, cols, vals)
    y2 = _linear2(p[0, :N_NODES], p[1, :N_NODES], W2, b2)
    q = _spmm_call(y2, rows, cols, vals)
    return _head(q[0, :N_NODES], q[1, :N_NODES], Wc, bc)


# speedup vs baseline: 1.0002x; 1.0002x over previous
import functools

import jax
import jax.numpy as jnp
from jax import lax
from jax.experimental import pallas as pl
from jax.experimental.pallas import tpu as pltpu
from jax.experimental.pallas import tpu_sc as plsc

N_NODES = 10000
IN_FEATS = 128
HIDDEN = 128
NUM_CLASSES = 64
N_EDGES = 320000

NC = 2
NS = 16
L = 16
NW = NC * NS
E_PER_W = N_EDGES // NW
CHUNK = 128
E_PER_W_PAD = 10240
N_CHUNKS = E_PER_W_PAD // CHUNK
E_TOTAL_PAD = NW * E_PER_W_PAD
N_PAD = 10112
ROWS_PER_TILE = N_PAD // NS
N_FEAT_REGS = HIDDEN // L


def _spmm_body(y_hbm, rows_hbm, cols_hbm, vals_hbm, out_hbm,
               rowb, colb, valb, gbuf, shared, semg0, semg1, semi):
    cid = lax.axis_index("c")
    tid = lax.axis_index("s")
    wid = cid * NS + tid

    zero = jnp.zeros((L,), jnp.float32)

    def zero_row(i, c):
        for j in range(N_FEAT_REGS):
            gbuf[0, i, pl.ds(j * L, L)] = zero
        return c

    lax.fori_loop(0, CHUNK, zero_row, 0)
    base = tid * ROWS_PER_TILE
    done = 0
    while done < ROWS_PER_TILE:
        n = min(CHUNK, ROWS_PER_TILE - done)
        pltpu.sync_copy(gbuf.at[0, pl.ds(0, n)],
                        shared.at[pl.ds(base + done, n)])
        done += n

    def stage_idx(k, b):
        ci = pltpu.async_copy(cols_hbm.at[wid, k], colb.at[b], semi)
        ri = pltpu.async_copy(rows_hbm.at[wid, k], rowb.at[b], semi)
        vi = pltpu.async_copy(vals_hbm.at[wid, k], valb.at[b], semi)
        ci.wait()
        ri.wait()
        vi.wait()

    def start_gather(b):
        sem = semg0 if b == 0 else semg1
        pltpu.async_copy(y_hbm.at[colb.at[b]], gbuf.at[b], sem)

    def wait_gather(b):
        sem = semg0 if b == 0 else semg1
        pltpu.make_async_copy(y_hbm.at[colb.at[b]], gbuf.at[b], sem).wait()

    def scale_scatter(b):
        def scale_group(g, c2):
            vvec = valb[b, pl.ds(g * L, L)]
            for li in range(L):
                v = vvec[li]
                i = g * L + li
                for j in range(N_FEAT_REGS):
                    sl = pl.ds(j * L, L)
                    gbuf[b, i, sl] = gbuf[b, i, sl] * v
            return c2

        lax.fori_loop(0, CHUNK // L, scale_group, 0)
        pltpu.sync_copy(gbuf.at[b], shared.at[rowb.at[b]], add=True)

    stage_idx(0, 0)
    start_gather(0)
    stage_idx(1, 1)
    plsc.subcore_barrier()

    def pair_body(p, c):
        k = 2 * p
        wait_gather(0)
        start_gather(1)
        scale_scatter(0)

        @pl.when(k + 2 < N_CHUNKS)
        def _():
            stage_idx(k + 2, 0)

        wait_gather(1)

        @pl.when(k + 2 < N_CHUNKS)
        def _():
            start_gather(0)

        scale_scatter(1)

        @pl.when(k + 3 < N_CHUNKS)
        def _():
            stage_idx(k + 3, 1)

        return c

    lax.fori_loop(0, N_CHUNKS // 2, pair_body, 0)

    plsc.subcore_barrier()
    pltpu.sync_copy(
        shared.at[pl.ds(base, ROWS_PER_TILE)],
        out_hbm.at[cid, pl.ds(base, ROWS_PER_TILE)])


_spmm_call = pl.kernel(
    _spmm_body,
    out_type=jax.ShapeDtypeStruct((NC, N_PAD, HIDDEN), jnp.float32),
    mesh=plsc.VectorSubcoreMesh(
        core_axis_name="c", subcore_axis_name="s",
        num_cores=NC, num_subcores=NS),
    scratch_types=[
        pltpu.VMEM((2, CHUNK), jnp.int32),
        pltpu.VMEM((2, CHUNK), jnp.int32),
        pltpu.VMEM((2, CHUNK), jnp.float32),
        pltpu.VMEM((2, CHUNK, HIDDEN), jnp.float32),
        pltpu.VMEM_SHARED((N_PAD, HIDDEN), jnp.float32),
        pltpu.SemaphoreType.DMA,
        pltpu.SemaphoreType.DMA,
        pltpu.SemaphoreType.DMA,
    ],
)


MBLK = 400


def _lin1_body(x_ref, w_ref, b_ref, o_ref):
    o_ref[...] = (
        jnp.dot(x_ref[...], w_ref[...], preferred_element_type=jnp.float32)
        + b_ref[...])


def _lin2_body(p0_ref, p1_ref, w_ref, b_ref, o_ref):
    h = jnp.maximum(p0_ref[...] + p1_ref[...], 0.0)
    o_ref[...] = (
        jnp.dot(h, w_ref[...], preferred_element_type=jnp.float32)
        + b_ref[...])


def _head_body(q0_ref, q1_ref, wc_ref, bc_ref, o_ref, acc_ref):
    i = pl.program_id(0)

    @pl.when(i == 0)
    def _():
        acc_ref[...] = jnp.zeros_like(acc_ref)

    h = jnp.maximum(q0_ref[...] + q1_ref[...], 0.0)
    acc_ref[...] += jnp.sum(h, axis=0, keepdims=True)

    @pl.when(i == pl.num_programs(0) - 1)
    def _():
        pooled = acc_ref[...] * (1.0 / N_NODES)
        o_ref[...] = (
            jnp.dot(pooled, wc_ref[...], preferred_element_type=jnp.float32)
            + bc_ref[...])


def _linear1(x, W, b):
    return pl.pallas_call(
        _lin1_body,
        grid=(N_NODES // MBLK,),
        in_specs=[
            pl.BlockSpec((MBLK, IN_FEATS), lambda i: (i, 0)),
            pl.BlockSpec((IN_FEATS, HIDDEN), lambda i: (0, 0)),
            pl.BlockSpec((1, HIDDEN), lambda i: (0, 0)),
        ],
        out_specs=pl.BlockSpec((MBLK, HIDDEN), lambda i: (i, 0)),
        out_shape=jax.ShapeDtypeStruct((N_NODES, HIDDEN), jnp.float32),
    )(x, W, b.reshape(1, HIDDEN))


def _linear2(p0, p1, W, b):
    return pl.pallas_call(
        _lin2_body,
        grid=(N_NODES // MBLK,),
        in_specs=[
            pl.BlockSpec((MBLK, HIDDEN), lambda i: (i, 0)),
            pl.BlockSpec((MBLK, HIDDEN), lambda i: (i, 0)),
            pl.BlockSpec((HIDDEN, HIDDEN), lambda i: (0, 0)),
            pl.BlockSpec((1, HIDDEN), lambda i: (0, 0)),
        ],
        out_specs=pl.BlockSpec((MBLK, HIDDEN), lambda i: (i, 0)),
        out_shape=jax.ShapeDtypeStruct((N_NODES, HIDDEN), jnp.float32),
    )(p0, p1, W, b.reshape(1, HIDDEN))


def _head(q0, q1, Wc, bc):
    out = pl.pallas_call(
        _head_body,
        grid=(N_NODES // MBLK,),
        in_specs=[
            pl.BlockSpec((MBLK, HIDDEN), lambda i: (i, 0)),
            pl.BlockSpec((MBLK, HIDDEN), lambda i: (i, 0)),
            pl.BlockSpec((HIDDEN, NUM_CLASSES), lambda i: (0, 0)),
            pl.BlockSpec((1, NUM_CLASSES), lambda i: (0, 0)),
        ],
        out_specs=pl.BlockSpec((1, NUM_CLASSES), lambda i: (0, 0)),
        out_shape=jax.ShapeDtypeStruct((1, NUM_CLASSES), jnp.float32),
        scratch_shapes=[pltpu.VMEM((1, HIDDEN), jnp.float32)],
    )(q0, q1, Wc, bc.reshape(1, NUM_CLASSES))
    return out.reshape(NUM_CLASSES)


@jax.jit
def kernel(x, adj_indices, adj_values, W1, b1, W2, b2, Wc, bc):
    pad = E_TOTAL_PAD - N_EDGES
    trash = N_NODES + (jnp.arange(pad, dtype=jnp.int32) % (N_PAD - N_NODES))
    rows = jnp.concatenate(
        [adj_indices[0].astype(jnp.int32), trash]).reshape(NW, N_CHUNKS, CHUNK)
    cols = jnp.concatenate(
        [adj_indices[1].astype(jnp.int32),
         jnp.zeros((pad,), jnp.int32)]).reshape(NW, N_CHUNKS, CHUNK)
    vals = jnp.concatenate(
        [adj_values, jnp.zeros((pad,), jnp.float32)]
    ).reshape(NW, N_CHUNKS, CHUNK)

    y1 = _linear1(x, W1, b1)
    p = _spmm_call(y1, rows, cols, vals)
    y2 = _linear2(p[0, :N_NODES], p[1, :N_NODES], W2, b2)
    q = _spmm_call(y2, rows, cols, vals)
    return _head(q[0, :N_NODES], q[1, :N_NODES], Wc, bc)


# speedup vs baseline: 1.3789x; 1.3787x over previous
import functools

import jax
import jax.numpy as jnp
from jax import lax
from jax.experimental import pallas as pl
from jax.experimental.pallas import tpu as pltpu
from jax.experimental.pallas import tpu_sc as plsc

N_NODES = 10000
IN_FEATS = 128
HIDDEN = 128
NUM_CLASSES = 64
N_EDGES = 320000

NC = 2
NS = 16
L = 16
NW = NC * NS
CHUNK = 128
NCH0 = 38
NCH1 = 120
TOTAL_CHUNKS = NS * (NCH0 + NCH1)
E_TOTAL_PAD = TOTAL_CHUNKS * CHUNK
N_PAD = 10112
ROWS_PER_TILE = N_PAD // NS
N_FEAT_REGS = HIDDEN // L


def _spmm_body(y_hbm, rows_hbm, cols_hbm, vals_hbm, out_hbm,
               rowb, colb, valb, gbuf, shared, semg0, semg1, semi):
    cid = lax.axis_index("c")
    tid = lax.axis_index("s")
    n_chunks = jnp.where(cid == 0, NCH0, NCH1)
    chunk_base = jnp.where(cid == 0, tid * NCH0, NS * NCH0 + tid * NCH1)

    zero = jnp.zeros((L,), jnp.float32)

    def zero_row(i, c):
        for j in range(N_FEAT_REGS):
            gbuf[0, i, pl.ds(j * L, L)] = zero
        return c

    lax.fori_loop(0, CHUNK, zero_row, 0)
    base = tid * ROWS_PER_TILE
    done = 0
    while done < ROWS_PER_TILE:
        n = min(CHUNK, ROWS_PER_TILE - done)
        pltpu.sync_copy(gbuf.at[0, pl.ds(0, n)],
                        shared.at[pl.ds(base + done, n)])
        done += n

    def stage_idx(k, b):
        kk = chunk_base + k
        ci = pltpu.async_copy(cols_hbm.at[kk], colb.at[b], semi)
        ri = pltpu.async_copy(rows_hbm.at[kk], rowb.at[b], semi)
        vi = pltpu.async_copy(vals_hbm.at[kk], valb.at[b], semi)
        ci.wait()
        ri.wait()
        vi.wait()

    def start_gather(b):
        sem = semg0 if b == 0 else semg1
        pltpu.async_copy(y_hbm.at[colb.at[b]], gbuf.at[b], sem)

    def wait_gather(b):
        sem = semg0 if b == 0 else semg1
        pltpu.make_async_copy(y_hbm.at[colb.at[b]], gbuf.at[b], sem).wait()

    def scale_scatter(b):
        def scale_group(g, c2):
            vvec = valb[b, pl.ds(g * L, L)]
            for li in range(L):
                v = vvec[li]
                i = g * L + li
                for j in range(N_FEAT_REGS):
                    sl = pl.ds(j * L, L)
                    gbuf[b, i, sl] = gbuf[b, i, sl] * v
            return c2

        lax.fori_loop(0, CHUNK // L, scale_group, 0)
        pltpu.sync_copy(gbuf.at[b], shared.at[rowb.at[b]], add=True)

    stage_idx(0, 0)
    start_gather(0)
    stage_idx(1, 1)
    plsc.subcore_barrier()

    def pair_body(p, c):
        k = 2 * p
        wait_gather(0)
        start_gather(1)
        scale_scatter(0)

        @pl.when(k + 2 < n_chunks)
        def _():
            stage_idx(k + 2, 0)

        wait_gather(1)

        @pl.when(k + 2 < n_chunks)
        def _():
            start_gather(0)

        scale_scatter(1)

        @pl.when(k + 3 < n_chunks)
        def _():
            stage_idx(k + 3, 1)

        return c

    lax.fori_loop(0, n_chunks // 2, pair_body, 0)

    plsc.subcore_barrier()
    pltpu.sync_copy(
        shared.at[pl.ds(base, ROWS_PER_TILE)],
        out_hbm.at[cid, pl.ds(base, ROWS_PER_TILE)])


_spmm_call = pl.kernel(
    _spmm_body,
    out_type=jax.ShapeDtypeStruct((NC, N_PAD, HIDDEN), jnp.float32),
    mesh=plsc.VectorSubcoreMesh(
        core_axis_name="c", subcore_axis_name="s",
        num_cores=NC, num_subcores=NS),
    scratch_types=[
        pltpu.VMEM((2, CHUNK), jnp.int32),
        pltpu.VMEM((2, CHUNK), jnp.int32),
        pltpu.VMEM((2, CHUNK), jnp.float32),
        pltpu.VMEM((2, CHUNK, HIDDEN), jnp.float32),
        pltpu.VMEM_SHARED((N_PAD, HIDDEN), jnp.float32),
        pltpu.SemaphoreType.DMA,
        pltpu.SemaphoreType.DMA,
        pltpu.SemaphoreType.DMA,
    ],
)


MBLK = 400


def _lin1_body(x_ref, w_ref, b_ref, o_ref):
    o_ref[...] = (
        jnp.dot(x_ref[...], w_ref[...], preferred_element_type=jnp.float32)
        + b_ref[...])


def _lin2_body(p0_ref, p1_ref, w_ref, b_ref, o_ref):
    h = jnp.maximum(p0_ref[...] + p1_ref[...], 0.0)
    o_ref[...] = (
        jnp.dot(h, w_ref[...], preferred_element_type=jnp.float32)
        + b_ref[...])


def _head_body(q0_ref, q1_ref, wc_ref, bc_ref, o_ref, acc_ref):
    i = pl.program_id(0)

    @pl.when(i == 0)
    def _():
        acc_ref[...] = jnp.zeros_like(acc_ref)

    h = jnp.maximum(q0_ref[...] + q1_ref[...], 0.0)
    acc_ref[...] += jnp.sum(h, axis=0, keepdims=True)

    @pl.when(i == pl.num_programs(0) - 1)
    def _():
        pooled = acc_ref[...] * (1.0 / N_NODES)
        o_ref[...] = (
            jnp.dot(pooled, wc_ref[...], preferred_element_type=jnp.float32)
            + bc_ref[...])


def _linear1(x, W, b):
    return pl.pallas_call(
        _lin1_body,
        grid=(N_NODES // MBLK,),
        in_specs=[
            pl.BlockSpec((MBLK, IN_FEATS), lambda i: (i, 0)),
            pl.BlockSpec((IN_FEATS, HIDDEN), lambda i: (0, 0)),
            pl.BlockSpec((1, HIDDEN), lambda i: (0, 0)),
        ],
        out_specs=pl.BlockSpec((MBLK, HIDDEN), lambda i: (i, 0)),
        out_shape=jax.ShapeDtypeStruct((N_NODES, HIDDEN), jnp.float32),
    )(x, W, b.reshape(1, HIDDEN))


def _linear2(p0, p1, W, b):
    return pl.pallas_call(
        _lin2_body,
        grid=(N_NODES // MBLK,),
        in_specs=[
            pl.BlockSpec((MBLK, HIDDEN), lambda i: (i, 0)),
            pl.BlockSpec((MBLK, HIDDEN), lambda i: (i, 0)),
            pl.BlockSpec((HIDDEN, HIDDEN), lambda i: (0, 0)),
            pl.BlockSpec((1, HIDDEN), lambda i: (0, 0)),
        ],
        out_specs=pl.BlockSpec((MBLK, HIDDEN), lambda i: (i, 0)),
        out_shape=jax.ShapeDtypeStruct((N_NODES, HIDDEN), jnp.float32),
    )(p0, p1, W, b.reshape(1, HIDDEN))


def _head(q0, q1, Wc, bc):
    out = pl.pallas_call(
        _head_body,
        grid=(N_NODES // MBLK,),
        in_specs=[
            pl.BlockSpec((MBLK, HIDDEN), lambda i: (i, 0)),
            pl.BlockSpec((MBLK, HIDDEN), lambda i: (i, 0)),
            pl.BlockSpec((HIDDEN, NUM_CLASSES), lambda i: (0, 0)),
            pl.BlockSpec((1, NUM_CLASSES), lambda i: (0, 0)),
        ],
        out_specs=pl.BlockSpec((1, NUM_CLASSES), lambda i: (0, 0)),
        out_shape=jax.ShapeDtypeStruct((1, NUM_CLASSES), jnp.float32),
        scratch_shapes=[pltpu.VMEM((1, HIDDEN), jnp.float32)],
    )(q0, q1, Wc, bc.reshape(1, NUM_CLASSES))
    return out.reshape(NUM_CLASSES)


@jax.jit
def kernel(x, adj_indices, adj_values, W1, b1, W2, b2, Wc, bc):
    pad = E_TOTAL_PAD - N_EDGES
    trash = N_NODES + (jnp.arange(pad, dtype=jnp.int32) % (N_PAD - N_NODES))
    rows = jnp.concatenate(
        [adj_indices[0].astype(jnp.int32), trash]
    ).reshape(TOTAL_CHUNKS, CHUNK)
    cols = jnp.concatenate(
        [adj_indices[1].astype(jnp.int32),
         jnp.zeros((pad,), jnp.int32)]).reshape(TOTAL_CHUNKS, CHUNK)
    vals = jnp.concatenate(
        [adj_values, jnp.zeros((pad,), jnp.float32)]
    ).reshape(TOTAL_CHUNKS, CHUNK)

    y1 = _linear1(x, W1, b1)
    p = _spmm_call(y1, rows, cols, vals)
    y2 = _linear2(p[0, :N_NODES], p[1, :N_NODES], W2, b2)
    q = _spmm_call(y2, rows, cols, vals)
    return _head(q[0, :N_NODES], q[1, :N_NODES], Wc, bc)


# speedup vs baseline: 1.8979x; 1.3764x over previous
import functools

import jax
import jax.numpy as jnp
from jax import lax
from jax.experimental import pallas as pl
from jax.experimental.pallas import tpu as pltpu
from jax.experimental.pallas import tpu_sc as plsc

N_NODES = 10000
IN_FEATS = 128
HIDDEN = 128
NUM_CLASSES = 64
N_EDGES = 320000

NC = 2
NS = 16
L = 16
NW = NC * NS
CHUNK = 128
NCH0 = 120
NCH1 = 38
TOTAL_CHUNKS = NS * (NCH0 + NCH1)
E_TOTAL_PAD = TOTAL_CHUNKS * CHUNK
N_PAD = 10112
ROWS_PER_TILE = N_PAD // NS
N_FEAT_REGS = HIDDEN // L


def _spmm_body(y_hbm, rows_hbm, cols_hbm, vals_hbm, out_hbm,
               rowb, colb, valb, gbuf, shared, semg0, semg1, semi):
    cid = lax.axis_index("c")
    tid = lax.axis_index("s")
    n_chunks = jnp.where(cid == 0, NCH0, NCH1)
    chunk_base = jnp.where(cid == 0, tid * NCH0, NS * NCH0 + tid * NCH1)

    zero = jnp.zeros((L,), jnp.float32)

    def zero_row(i, c):
        for j in range(N_FEAT_REGS):
            gbuf[0, i, pl.ds(j * L, L)] = zero
        return c

    lax.fori_loop(0, CHUNK, zero_row, 0)
    base = tid * ROWS_PER_TILE
    done = 0
    while done < ROWS_PER_TILE:
        n = min(CHUNK, ROWS_PER_TILE - done)
        pltpu.sync_copy(gbuf.at[0, pl.ds(0, n)],
                        shared.at[pl.ds(base + done, n)])
        done += n

    def stage_idx(k, b):
        kk = chunk_base + k
        ci = pltpu.async_copy(cols_hbm.at[kk], colb.at[b], semi)
        ri = pltpu.async_copy(rows_hbm.at[kk], rowb.at[b], semi)
        vi = pltpu.async_copy(vals_hbm.at[kk], valb.at[b], semi)
        ci.wait()
        ri.wait()
        vi.wait()

    def start_gather(b):
        sem = semg0 if b == 0 else semg1
        pltpu.async_copy(y_hbm.at[colb.at[b]], gbuf.at[b], sem)

    def wait_gather(b):
        sem = semg0 if b == 0 else semg1
        pltpu.make_async_copy(y_hbm.at[colb.at[b]], gbuf.at[b], sem).wait()

    def scale_scatter(b):
        def scale_group(g, c2):
            vvec = valb[b, pl.ds(g * L, L)]
            for li in range(L):
                v = vvec[li]
                i = g * L + li
                for j in range(N_FEAT_REGS):
                    sl = pl.ds(j * L, L)
                    gbuf[b, i, sl] = gbuf[b, i, sl] * v
            return c2

        lax.fori_loop(0, CHUNK // L, scale_group, 0)
        pltpu.sync_copy(gbuf.at[b], shared.at[rowb.at[b]], add=True)

    stage_idx(0, 0)
    start_gather(0)
    stage_idx(1, 1)
    plsc.subcore_barrier()

    def pair_body(p, c):
        k = 2 * p
        wait_gather(0)
        start_gather(1)
        scale_scatter(0)

        @pl.when(k + 2 < n_chunks)
        def _():
            stage_idx(k + 2, 0)

        wait_gather(1)

        @pl.when(k + 2 < n_chunks)
        def _():
            start_gather(0)

        scale_scatter(1)

        @pl.when(k + 3 < n_chunks)
        def _():
            stage_idx(k + 3, 1)

        return c

    lax.fori_loop(0, n_chunks // 2, pair_body, 0)

    plsc.subcore_barrier()
    pltpu.sync_copy(
        shared.at[pl.ds(base, ROWS_PER_TILE)],
        out_hbm.at[cid, pl.ds(base, ROWS_PER_TILE)])


_spmm_call = pl.kernel(
    _spmm_body,
    out_type=jax.ShapeDtypeStruct((NC, N_PAD, HIDDEN), jnp.float32),
    mesh=plsc.VectorSubcoreMesh(
        core_axis_name="c", subcore_axis_name="s",
        num_cores=NC, num_subcores=NS),
    scratch_types=[
        pltpu.VMEM((2, CHUNK), jnp.int32),
        pltpu.VMEM((2, CHUNK), jnp.int32),
        pltpu.VMEM((2, CHUNK), jnp.float32),
        pltpu.VMEM((2, CHUNK, HIDDEN), jnp.float32),
        pltpu.VMEM_SHARED((N_PAD, HIDDEN), jnp.float32),
        pltpu.SemaphoreType.DMA,
        pltpu.SemaphoreType.DMA,
        pltpu.SemaphoreType.DMA,
    ],
)


MBLK = 400


def _lin1_body(x_ref, w_ref, b_ref, o_ref):
    o_ref[...] = (
        jnp.dot(x_ref[...], w_ref[...], preferred_element_type=jnp.float32)
        + b_ref[...])


def _lin2_body(p0_ref, p1_ref, w_ref, b_ref, o_ref):
    h = jnp.maximum(p0_ref[...] + p1_ref[...], 0.0)
    o_ref[...] = (
        jnp.dot(h, w_ref[...], preferred_element_type=jnp.float32)
        + b_ref[...])


def _head_body(q0_ref, q1_ref, wc_ref, bc_ref, o_ref, acc_ref):
    i = pl.program_id(0)

    @pl.when(i == 0)
    def _():
        acc_ref[...] = jnp.zeros_like(acc_ref)

    h = jnp.maximum(q0_ref[...] + q1_ref[...], 0.0)
    acc_ref[...] += jnp.sum(h, axis=0, keepdims=True)

    @pl.when(i == pl.num_programs(0) - 1)
    def _():
        pooled = acc_ref[...] * (1.0 / N_NODES)
        o_ref[...] = (
            jnp.dot(pooled, wc_ref[...], preferred_element_type=jnp.float32)
            + bc_ref[...])


def _linear1(x, W, b):
    return pl.pallas_call(
        _lin1_body,
        grid=(N_NODES // MBLK,),
        in_specs=[
            pl.BlockSpec((MBLK, IN_FEATS), lambda i: (i, 0)),
            pl.BlockSpec((IN_FEATS, HIDDEN), lambda i: (0, 0)),
            pl.BlockSpec((1, HIDDEN), lambda i: (0, 0)),
        ],
        out_specs=pl.BlockSpec((MBLK, HIDDEN), lambda i: (i, 0)),
        out_shape=jax.ShapeDtypeStruct((N_NODES, HIDDEN), jnp.float32),
    )(x, W, b.reshape(1, HIDDEN))


def _linear2(p0, p1, W, b):
    return pl.pallas_call(
        _lin2_body,
        grid=(N_NODES // MBLK,),
        in_specs=[
            pl.BlockSpec((MBLK, HIDDEN), lambda i: (i, 0)),
            pl.BlockSpec((MBLK, HIDDEN), lambda i: (i, 0)),
            pl.BlockSpec((HIDDEN, HIDDEN), lambda i: (0, 0)),
            pl.BlockSpec((1, HIDDEN), lambda i: (0, 0)),
        ],
        out_specs=pl.BlockSpec((MBLK, HIDDEN), lambda i: (i, 0)),
        out_shape=jax.ShapeDtypeStruct((N_NODES, HIDDEN), jnp.float32),
    )(p0, p1, W, b.reshape(1, HIDDEN))


def _head(q0, q1, Wc, bc):
    out = pl.pallas_call(
        _head_body,
        grid=(N_NODES // MBLK,),
        in_specs=[
            pl.BlockSpec((MBLK, HIDDEN), lambda i: (i, 0)),
            pl.BlockSpec((MBLK, HIDDEN), lambda i: (i, 0)),
            pl.BlockSpec((HIDDEN, NUM_CLASSES), lambda i: (0, 0)),
            pl.BlockSpec((1, NUM_CLASSES), lambda i: (0, 0)),
        ],
        out_specs=pl.BlockSpec((1, NUM_CLASSES), lambda i: (0, 0)),
        out_shape=jax.ShapeDtypeStruct((1, NUM_CLASSES), jnp.float32),
        scratch_shapes=[pltpu.VMEM((1, HIDDEN), jnp.float32)],
    )(q0, q1, Wc, bc.reshape(1, NUM_CLASSES))
    return out.reshape(NUM_CLASSES)


@jax.jit
def kernel(x, adj_indices, adj_values, W1, b1, W2, b2, Wc, bc):
    pad = E_TOTAL_PAD - N_EDGES
    trash = N_NODES + (jnp.arange(pad, dtype=jnp.int32) % (N_PAD - N_NODES))
    rows = jnp.concatenate(
        [adj_indices[0].astype(jnp.int32), trash]
    ).reshape(TOTAL_CHUNKS, CHUNK)
    cols = jnp.concatenate(
        [adj_indices[1].astype(jnp.int32),
         jnp.zeros((pad,), jnp.int32)]).reshape(TOTAL_CHUNKS, CHUNK)
    vals = jnp.concatenate(
        [adj_values, jnp.zeros((pad,), jnp.float32)]
    ).reshape(TOTAL_CHUNKS, CHUNK)

    y1 = _linear1(x, W1, b1)
    p = _spmm_call(y1, rows, cols, vals)
    y2 = _linear2(p[0, :N_NODES], p[1, :N_NODES], W2, b2)
    q = _spmm_call(y2, rows, cols, vals)
    return _head(q[0, :N_NODES], q[1, :N_NODES], Wc, bc)


# speedup vs baseline: 1.9010x; 1.0016x over previous
import jax
import jax.numpy as jnp
from jax import lax
from jax.experimental import pallas as pl
from jax.experimental.pallas import tpu as pltpu
from jax.experimental.pallas import tpu_sc as plsc

N_NODES = 10000
IN_FEATS = 128
HIDDEN = 128
NUM_CLASSES = 64
N_EDGES = 320000

NC = 2
NS = 16
L = 16
NW = NC * NS
CHUNK = 128
NCH0 = 120
NCH1 = 38
TOTAL_CHUNKS = NS * (NCH0 + NCH1)
E_TOTAL_PAD = TOTAL_CHUNKS * CHUNK
N_PAD = 10112
ROWS_PER_TILE = N_PAD // NS
N_FEAT_REGS = HIDDEN // L


def _spmm_body(y_hbm, rows_hbm, cols_hbm, vals_hbm, out_hbm,
               rowb, colb, valb, gbuf, shared, semg0, semg1, semi):
    cid = lax.axis_index("c")
    tid = lax.axis_index("s")
    n_chunks = jnp.where(cid == 0, NCH0, NCH1)
    chunk_base = jnp.where(cid == 0, tid * NCH0, NS * NCH0 + tid * NCH1)

    zero = jnp.zeros((L,), jnp.float32)

    def zero_row(i, c):
        for j in range(N_FEAT_REGS):
            gbuf[0, i, pl.ds(j * L, L)] = zero
        return c

    lax.fori_loop(0, CHUNK, zero_row, 0)
    base = tid * ROWS_PER_TILE
    done = 0
    while done < ROWS_PER_TILE:
        n = min(CHUNK, ROWS_PER_TILE - done)
        pltpu.sync_copy(gbuf.at[0, pl.ds(0, n)],
                        shared.at[pl.ds(base + done, n)])
        done += n

    def stage_idx(k, b):
        kk = chunk_base + k
        ci = pltpu.async_copy(cols_hbm.at[kk], colb.at[b], semi)
        ri = pltpu.async_copy(rows_hbm.at[kk], rowb.at[b], semi)
        vi = pltpu.async_copy(vals_hbm.at[kk], valb.at[b], semi)
        ci.wait()
        ri.wait()
        vi.wait()

    def start_gather(b):
        sem = semg0 if b == 0 else semg1
        pltpu.async_copy(y_hbm.at[colb.at[b]], gbuf.at[b], sem)

    def wait_gather(b):
        sem = semg0 if b == 0 else semg1
        pltpu.make_async_copy(y_hbm.at[colb.at[b]], gbuf.at[b], sem).wait()

    def scale_scatter(b):
        def scale_group(g, c2):
            vvec = valb[b, pl.ds(g * L, L)]
            for li in range(L):
                v = vvec[li]
                i = g * L + li
                for j in range(N_FEAT_REGS):
                    sl = pl.ds(j * L, L)
                    gbuf[b, i, sl] = gbuf[b, i, sl] * v
            return c2

        lax.fori_loop(0, CHUNK // L, scale_group, 0)
        pltpu.sync_copy(gbuf.at[b], shared.at[rowb.at[b]], add=True)

    stage_idx(0, 0)
    start_gather(0)
    stage_idx(1, 1)
    plsc.subcore_barrier()

    def pair_body(p, c):
        k = 2 * p
        wait_gather(0)
        start_gather(1)
        scale_scatter(0)

        @pl.when(k + 2 < n_chunks)
        def _():
            stage_idx(k + 2, 0)

        wait_gather(1)

        @pl.when(k + 2 < n_chunks)
        def _():
            start_gather(0)

        scale_scatter(1)

        @pl.when(k + 3 < n_chunks)
        def _():
            stage_idx(k + 3, 1)

        return c

    lax.fori_loop(0, n_chunks // 2, pair_body, 0)

    plsc.subcore_barrier()
    pltpu.sync_copy(
        shared.at[pl.ds(base, ROWS_PER_TILE)],
        out_hbm.at[cid, pl.ds(base, ROWS_PER_TILE)])


_spmm_call = pl.kernel(
    _spmm_body,
    out_type=jax.ShapeDtypeStruct((NC, N_PAD, HIDDEN), jnp.float32),
    mesh=plsc.VectorSubcoreMesh(
        core_axis_name="c", subcore_axis_name="s",
        num_cores=NC, num_subcores=NS),
    scratch_types=[
        pltpu.VMEM((2, CHUNK), jnp.int32),
        pltpu.VMEM((2, CHUNK), jnp.int32),
        pltpu.VMEM((2, CHUNK), jnp.float32),
        pltpu.VMEM((2, CHUNK, HIDDEN), jnp.float32),
        pltpu.VMEM_SHARED((N_PAD, HIDDEN), jnp.float32),
        pltpu.SemaphoreType.DMA,
        pltpu.SemaphoreType.DMA,
        pltpu.SemaphoreType.DMA,
    ],
)


MBLK = 400


def _lin1_body(x_ref, w_ref, b_ref, o_ref):
    o_ref[...] = (
        jnp.dot(x_ref[...], w_ref[...], preferred_element_type=jnp.float32)
        + b_ref[...])


def _lin2_body(p0_ref, p1_ref, w_ref, b_ref, o_ref):
    h = jnp.maximum(p0_ref[...] + p1_ref[...], 0.0)
    o_ref[...] = (
        jnp.dot(h, w_ref[...], preferred_element_type=jnp.float32)
        + b_ref[...])


def _head_body(q0_ref, q1_ref, wc_ref, bc_ref, o_ref, acc_ref):
    i = pl.program_id(0)

    @pl.when(i == 0)
    def _():
        acc_ref[...] = jnp.zeros_like(acc_ref)

    h = jnp.maximum(q0_ref[...] + q1_ref[...], 0.0)
    acc_ref[...] += jnp.sum(h, axis=0, keepdims=True)

    @pl.when(i == pl.num_programs(0) - 1)
    def _():
        pooled = acc_ref[...] * (1.0 / N_NODES)
        o_ref[...] = (
            jnp.dot(pooled, wc_ref[...], preferred_element_type=jnp.float32)
            + bc_ref[...])


def _linear1(x, W, b):
    return pl.pallas_call(
        _lin1_body,
        grid=(N_NODES // MBLK,),
        in_specs=[
            pl.BlockSpec((MBLK, IN_FEATS), lambda i: (i, 0)),
            pl.BlockSpec((IN_FEATS, HIDDEN), lambda i: (0, 0)),
            pl.BlockSpec((1, HIDDEN), lambda i: (0, 0)),
        ],
        out_specs=pl.BlockSpec((MBLK, HIDDEN), lambda i: (i, 0)),
        out_shape=jax.ShapeDtypeStruct((N_NODES, HIDDEN), jnp.float32),
    )(x, W, b.reshape(1, HIDDEN))


def _linear2(p0, p1, W, b):
    return pl.pallas_call(
        _lin2_body,
        grid=(N_NODES // MBLK,),
        in_specs=[
            pl.BlockSpec((MBLK, HIDDEN), lambda i: (i, 0)),
            pl.BlockSpec((MBLK, HIDDEN), lambda i: (i, 0)),
            pl.BlockSpec((HIDDEN, HIDDEN), lambda i: (0, 0)),
            pl.BlockSpec((1, HIDDEN), lambda i: (0, 0)),
        ],
        out_specs=pl.BlockSpec((MBLK, HIDDEN), lambda i: (i, 0)),
        out_shape=jax.ShapeDtypeStruct((N_NODES, HIDDEN), jnp.float32),
    )(p0, p1, W, b.reshape(1, HIDDEN))


def _head(q0, q1, Wc, bc):
    out = pl.pallas_call(
        _head_body,
        grid=(N_NODES // MBLK,),
        in_specs=[
            pl.BlockSpec((MBLK, HIDDEN), lambda i: (i, 0)),
            pl.BlockSpec((MBLK, HIDDEN), lambda i: (i, 0)),
            pl.BlockSpec((HIDDEN, NUM_CLASSES), lambda i: (0, 0)),
            pl.BlockSpec((1, NUM_CLASSES), lambda i: (0, 0)),
        ],
        out_specs=pl.BlockSpec((1, NUM_CLASSES), lambda i: (0, 0)),
        out_shape=jax.ShapeDtypeStruct((1, NUM_CLASSES), jnp.float32),
        scratch_shapes=[pltpu.VMEM((1, HIDDEN), jnp.float32)],
    )(q0, q1, Wc, bc.reshape(1, NUM_CLASSES))
    return out.reshape(NUM_CLASSES)


@jax.jit
def kernel(x, adj_indices, adj_values, W1, b1, W2, b2, Wc, bc):
    pad = E_TOTAL_PAD - N_EDGES
    trash = N_NODES + (jnp.arange(pad, dtype=jnp.int32) % (N_PAD - N_NODES))
    rows = jnp.concatenate(
        [adj_indices[0].astype(jnp.int32), trash]
    ).reshape(TOTAL_CHUNKS, CHUNK)
    cols = jnp.concatenate(
        [adj_indices[1].astype(jnp.int32),
         jnp.zeros((pad,), jnp.int32)]).reshape(TOTAL_CHUNKS, CHUNK)
    vals = jnp.concatenate(
        [adj_values, jnp.zeros((pad,), jnp.float32)]
    ).reshape(TOTAL_CHUNKS, CHUNK)

    y1 = _linear1(x, W1, b1)
    p = _spmm_call(y1, rows, cols, vals)
    y2 = _linear2(p[0, :N_NODES], p[1, :N_NODES], W2, b2)
    q = _spmm_call(y2, rows, cols, vals)
    return _head(q[0, :N_NODES], q[1, :N_NODES], Wc, bc)
